# baseline (speedup 1.0000x reference)
import functools

import jax
import jax.numpy as jnp
from jax import lax
from jax.experimental import pallas as pl
from jax.experimental.pallas import tpu as pltpu
from jax.experimental.pallas import tpu_sc as plsc

M_ROWS = 1_000_000
C = 128
B = 16384

NC = 2
NS = 16
L = 16
NW = NC * NS
B_PER_W = B // NW
CHUNK = 64
CHUNKS = tuple((o, CHUNK) for o in range(0, B_PER_W, CHUNK))
NBUF = 6
C_VECS = C // L

_mesh = plsc.VectorSubcoreMesh(
    core_axis_name="c", subcore_axis_name="s", num_cores=NC, num_subcores=NS
)


@functools.partial(
    pl.kernel,
    out_type=jax.ShapeDtypeStruct((NW, L), jnp.float32),
    mesh=_mesh,
    compiler_params=pltpu.CompilerParams(
        needs_layout_passes=False,
        disable_bounds_checks=True,
        disable_semaphore_checks=True,
        skip_device_barrier=True,
        use_tc_tiling_on_sc=False,
    ),
    scratch_types=[
        pltpu.VMEM((B_PER_W,), jnp.int32),
        pltpu.VMEM((NBUF, CHUNK, C), jnp.float32),
        pltpu.VMEM((NBUF, CHUNK, C), jnp.float32),
        pltpu.VMEM((L,), jnp.float32),
    ] + [pltpu.SemaphoreType.DMA] * (2 * NBUF),
)
def _sc_gather_dot(outputs_hbm, idx_hbm, conf_hbm, out_hbm,
                   idx_v, rows_v, outs_v, acc_v, *sems):
    wid = lax.axis_index("s") * NC + lax.axis_index("c")
    base = wid * B_PER_W
    gsems = sems[:NBUF]
    osems = sems[NBUF:]

    pltpu.sync_copy(idx_hbm.at[pl.ds(base, B_PER_W)], idx_v)

    def start(g):
        slot = g % NBUF
        off, rows = CHUNKS[g]
        gather = pltpu.async_copy(
            conf_hbm.at[idx_v.at[pl.ds(off, rows)]],
            rows_v.at[slot].at[pl.ds(0, rows)],
            gsems[slot])
        ocopy = pltpu.async_copy(
            outputs_hbm.at[pl.ds(base + off, rows)],
            outs_v.at[slot].at[pl.ds(0, rows)],
            osems[slot])
        return gather, ocopy

    pending = {g: start(g) for g in range(NBUF)}

    accs = tuple(jnp.zeros((L,), jnp.float32) for _ in range(C_VECS))
    for g in range(len(CHUNKS)):
        slot = g % NBUF
        rows = CHUNKS[g][1]
        gather, ocopy = pending.pop(g)
        gather.wait()
        ocopy.wait()

        def body(r, accs, slot=slot):
            return tuple(
                accs[cc]
                + rows_v[slot, r, pl.ds(cc * L, L)]
                * outs_v[slot, r, pl.ds(cc * L, L)]
                for cc in range(C_VECS)
            )
        accs = lax.fori_loop(0, rows, body, accs)
        if g + NBUF < len(CHUNKS):
            pending[g + NBUF] = start(g + NBUF)

    total = accs[0]
    for cc in range(1, C_VECS):
        total = total + accs[cc]
    acc_v[...] = total
    pltpu.sync_copy(acc_v, out_hbm.at[wid])


TC_BLOCK = 2048
N_TC_BLOCKS = B // TC_BLOCK


def _tc_body(x_ref, out_ref):
    i = pl.program_id(0)
    x = x_ref[...]
    s = jnp.sum(jnp.maximum(x, 0.0) + jnp.log1p(jnp.exp(-jnp.abs(x))))

    @pl.when(i == 0)
    def _init():
        out_ref[0, 0] = 0.0

    out_ref[0, 0] += s


_tc_dense = pl.pallas_call(
    _tc_body,
    grid=(N_TC_BLOCKS,),
    in_specs=[pl.BlockSpec((TC_BLOCK, C), lambda i: (i, 0))],
    out_specs=pl.BlockSpec(memory_space=pltpu.SMEM),
    out_shape=jax.ShapeDtypeStruct((1, 1), jnp.float32),
)


def _tc_combine_body(dense_ref, part_ref, out_ref):
    out_ref[0, 0] = (dense_ref[0, 0] - jnp.sum(part_ref[...])) * (1.0 / (B * C))


_tc_combine = pl.pallas_call(
    _tc_combine_body,
    in_specs=[
        pl.BlockSpec(memory_space=pltpu.SMEM),
        pl.BlockSpec((NW, L), lambda: (0, 0)),
    ],
    out_specs=pl.BlockSpec(memory_space=pltpu.SMEM),
    out_shape=jax.ShapeDtypeStruct((1, 1), jnp.float32),
)


def kernel(outputs, index, confidence):
    partials = _sc_gather_dot(outputs, index, confidence)
    dense = _tc_dense(outputs)
    return _tc_combine(dense, partials)[0, 0]


# speedup vs baseline: 1.0321x; 1.0321x over previous
import functools

import jax
import jax.numpy as jnp
from jax import lax
from jax.experimental import pallas as pl
from jax.experimental.pallas import tpu as pltpu
from jax.experimental.pallas import tpu_sc as plsc

M_ROWS = 1_000_000
C = 128
B = 16384

NC = 2
NS = 16
L = 16
NW = NC * NS
B_PER_W = B // NW
CHUNK = 64
CHUNKS = tuple((o, CHUNK) for o in range(0, B_PER_W, CHUNK))
NBUF = 6
C_VECS = C // L

_mesh = plsc.VectorSubcoreMesh(
    core_axis_name="c", subcore_axis_name="s", num_cores=NC, num_subcores=NS
)


@functools.partial(
    pl.kernel,
    out_type=jax.ShapeDtypeStruct((NW, L), jnp.float32),
    mesh=_mesh,
    compiler_params=pltpu.CompilerParams(
        needs_layout_passes=False,
        disable_bounds_checks=True,
        disable_semaphore_checks=True,
        skip_device_barrier=True,
    ),
    scratch_types=[
        pltpu.VMEM((B_PER_W,), jnp.int32),
        pltpu.VMEM((NBUF, CHUNK, C), jnp.float32),
        pltpu.VMEM((NBUF, CHUNK, C), jnp.float32),
        pltpu.VMEM((L,), jnp.float32),
    ] + [pltpu.SemaphoreType.DMA] * (2 * NBUF),
)
def _sc_gather_dot(outputs_hbm, idx_hbm, conf_hbm, out_hbm,
                   idx_v, rows_v, outs_v, acc_v, *sems):
    wid = lax.axis_index("s") * NC + lax.axis_index("c")
    base = wid * B_PER_W
    gsems = sems[:NBUF]
    osems = sems[NBUF:]

    pltpu.sync_copy(idx_hbm.at[pl.ds(base, B_PER_W)], idx_v)

    def start(g):
        slot = g % NBUF
        off, rows = CHUNKS[g]
        gather = pltpu.async_copy(
            conf_hbm.at[idx_v.at[pl.ds(off, rows)]],
            rows_v.at[slot].at[pl.ds(0, rows)],
            gsems[slot])
        ocopy = pltpu.async_copy(
            outputs_hbm.at[pl.ds(base + off, rows)],
            outs_v.at[slot].at[pl.ds(0, rows)],
            osems[slot])
        return gather, ocopy

    pending = {g: start(g) for g in range(NBUF)}

    accs = tuple(jnp.zeros((L,), jnp.float32) for _ in range(C_VECS))
    for g in range(len(CHUNKS)):
        slot = g % NBUF
        rows = CHUNKS[g][1]
        gather, ocopy = pending.pop(g)
        gather.wait()
        ocopy.wait()

        def body(r, accs, slot=slot):
            return tuple(
                accs[cc]
                + rows_v[slot, r, pl.ds(cc * L, L)]
                * outs_v[slot, r, pl.ds(cc * L, L)]
                for cc in range(C_VECS)
            )
        accs = lax.fori_loop(0, rows, body, accs)
        if g + NBUF < len(CHUNKS):
            pending[g + NBUF] = start(g + NBUF)

    total = accs[0]
    for cc in range(1, C_VECS):
        total = total + accs[cc]
    acc_v[...] = total
    pltpu.sync_copy(acc_v, out_hbm.at[wid])


TC_BLOCK = 2048
N_TC_BLOCKS = B // TC_BLOCK


def _tc_body(x_ref, out_ref):
    i = pl.program_id(0)
    x = x_ref[...]
    s = jnp.sum(jnp.maximum(x, 0.0) + jnp.log1p(jnp.exp(-jnp.abs(x))))

    @pl.when(i == 0)
    def _init():
        out_ref[0, 0] = 0.0

    out_ref[0, 0] += s


_tc_dense = pl.pallas_call(
    _tc_body,
    grid=(N_TC_BLOCKS,),
    in_specs=[pl.BlockSpec((TC_BLOCK, C), lambda i: (i, 0))],
    out_specs=pl.BlockSpec(memory_space=pltpu.SMEM),
    out_shape=jax.ShapeDtypeStruct((1, 1), jnp.float32),
)


def _tc_combine_body(dense_ref, part_ref, out_ref):
    out_ref[0, 0] = (dense_ref[0, 0] - jnp.sum(part_ref[...])) * (1.0 / (B * C))


_tc_combine = pl.pallas_call(
    _tc_combine_body,
    in_specs=[
        pl.BlockSpec(memory_space=pltpu.SMEM),
        pl.BlockSpec((NW, L), lambda: (0, 0)),
    ],
    out_specs=pl.BlockSpec(memory_space=pltpu.SMEM),
    out_shape=jax.ShapeDtypeStruct((1, 1), jnp.float32),
)


def kernel(outputs, index, confidence):
    partials = _sc_gather_dot(outputs, index, confidence)
    dense = _tc_dense(outputs)
    return _tc_combine(dense, partials)[0, 0]


# speedup vs baseline: 1.0504x; 1.0177x over previous
import functools

import jax
import jax.numpy as jnp
from jax import lax
from jax.experimental import pallas as pl
from jax.experimental.pallas import tpu as pltpu
from jax.experimental.pallas import tpu_sc as plsc

M_ROWS = 1_000_000
C = 128
B = 16384

NC = 2
NS = 16
L = 16
NW = NC * NS
B_PER_W = B // NW
CHUNK = 64
CHUNKS = tuple((o, CHUNK) for o in range(0, B_PER_W, CHUNK))
NBUF = 6
C_VECS = C // L

_mesh = plsc.VectorSubcoreMesh(
    core_axis_name="c", subcore_axis_name="s", num_cores=NC, num_subcores=NS
)


@functools.partial(
    pl.kernel,
    out_type=jax.ShapeDtypeStruct((NW, L), jnp.float32),
    mesh=_mesh,
    compiler_params=pltpu.CompilerParams(
        needs_layout_passes=False,
        disable_bounds_checks=True,
        disable_semaphore_checks=True,
        skip_device_barrier=True,
    ),
    scratch_types=[
        pltpu.VMEM((B_PER_W,), jnp.int32),
        pltpu.VMEM((NBUF, CHUNK, C), jnp.float32),
        pltpu.VMEM((NBUF, CHUNK, C), jnp.float32),
        pltpu.VMEM((L,), jnp.float32),
    ] + [pltpu.SemaphoreType.DMA] * (2 * NBUF),
)
def _sc_gather_dot(outputs_hbm, idx_hbm, conf_hbm, out_hbm,
                   idx_v, rows_v, outs_v, acc_v, *sems):
    wid = lax.axis_index("s") * NC + lax.axis_index("c")
    base = wid * B_PER_W
    gsems = sems[:NBUF]
    osems = sems[NBUF:]

    pltpu.sync_copy(idx_hbm.at[pl.ds(base, B_PER_W)], idx_v)

    def start(g):
        slot = g % NBUF
        off, rows = CHUNKS[g]
        gather = pltpu.async_copy(
            conf_hbm.at[idx_v.at[pl.ds(off, rows)]],
            rows_v.at[slot].at[pl.ds(0, rows)],
            gsems[slot])
        ocopy = pltpu.async_copy(
            outputs_hbm.at[pl.ds(base + off, rows)],
            outs_v.at[slot].at[pl.ds(0, rows)],
            osems[slot])
        return gather, ocopy

    pending = {g: start(g) for g in range(NBUF)}

    accs = tuple(jnp.zeros((L,), jnp.float32) for _ in range(C_VECS))
    for g in range(len(CHUNKS)):
        slot = g % NBUF
        rows = CHUNKS[g][1]
        gather, ocopy = pending.pop(g)
        gather.wait()
        ocopy.wait()

        def body(r, accs, slot=slot):
            return tuple(
                accs[cc]
                + rows_v[slot, r, pl.ds(cc * L, L)]
                * outs_v[slot, r, pl.ds(cc * L, L)]
                for cc in range(C_VECS)
            )
        accs = lax.fori_loop(0, rows, body, accs)
        if g + NBUF < len(CHUNKS):
            pending[g + NBUF] = start(g + NBUF)

    total = accs[0]
    for cc in range(1, C_VECS):
        total = total + accs[cc]
    acc_v[...] = total
    pltpu.sync_copy(acc_v, out_hbm.at[wid])


TC_BLOCK = 4096
N_TC_BLOCKS = B // TC_BLOCK


def _tc_body(x_ref, out_ref):
    i = pl.program_id(0)
    x = x_ref[...]
    s = jnp.sum(jnp.maximum(x, 0.0) + jnp.log1p(jnp.exp(-jnp.abs(x))))

    @pl.when(i == 0)
    def _init():
        out_ref[0, 0] = 0.0

    out_ref[0, 0] += s


_tc_dense = pl.pallas_call(
    _tc_body,
    grid=(N_TC_BLOCKS,),
    in_specs=[pl.BlockSpec((TC_BLOCK, C), lambda i: (i, 0))],
    out_specs=pl.BlockSpec(memory_space=pltpu.SMEM),
    out_shape=jax.ShapeDtypeStruct((1, 1), jnp.float32),
)


def _tc_combine_body(dense_ref, part_ref, out_ref):
    out_ref[0, 0] = (dense_ref[0, 0] - jnp.sum(part_ref[...])) * (1.0 / (B * C))


_tc_combine = pl.pallas_call(
    _tc_combine_body,
    in_specs=[
        pl.BlockSpec(memory_space=pltpu.SMEM),
        pl.BlockSpec((NW, L), lambda: (0, 0)),
    ],
    out_specs=pl.BlockSpec(memory_space=pltpu.SMEM),
    out_shape=jax.ShapeDtypeStruct((1, 1), jnp.float32),
)


def kernel(outputs, index, confidence):
    partials = _sc_gather_dot(outputs, index, confidence)
    dense = _tc_dense(outputs)
    return _tc_combine(dense, partials)[0, 0]


# speedup vs baseline: 1.0692x; 1.0179x over previous
import functools

import jax
import jax.numpy as jnp
from jax import lax
from jax.experimental import pallas as pl
from jax.experimental.pallas import tpu as pltpu
from jax.experimental.pallas import tpu_sc as plsc

M_ROWS = 1_000_000
C = 128
B = 16384

NC = 2
NS = 16
L = 16
NW = NC * NS
B_PER_W = B // NW
CHUNK = 64
CHUNKS = tuple((o, CHUNK) for o in range(0, B_PER_W, CHUNK))
NBUF = 6
C_VECS = C // L

_mesh = plsc.VectorSubcoreMesh(
    core_axis_name="c", subcore_axis_name="s", num_cores=NC, num_subcores=NS
)


@functools.partial(
    pl.kernel,
    out_type=jax.ShapeDtypeStruct((NW, L), jnp.float32),
    mesh=_mesh,
    compiler_params=pltpu.CompilerParams(
        needs_layout_passes=False,
        disable_bounds_checks=True,
        disable_semaphore_checks=True,
        skip_device_barrier=True,
    ),
    scratch_types=[
        pltpu.VMEM((B_PER_W,), jnp.int32),
        pltpu.VMEM((NBUF, CHUNK, C), jnp.float32),
        pltpu.VMEM((NBUF, CHUNK, C), jnp.float32),
        pltpu.VMEM((L,), jnp.float32),
    ] + [pltpu.SemaphoreType.DMA] * (2 * NBUF),
)
def _sc_gather_dot(outputs_hbm, idx_hbm, conf_hbm, out_hbm,
                   idx_v, rows_v, outs_v, acc_v, *sems):
    wid = lax.axis_index("s") * NC + lax.axis_index("c")
    base = wid * B_PER_W
    gsems = sems[:NBUF]
    osems = sems[NBUF:]

    pltpu.sync_copy(idx_hbm.at[pl.ds(base, B_PER_W)], idx_v)

    def start(g):
        slot = g % NBUF
        off, rows = CHUNKS[g]
        gather = pltpu.async_copy(
            conf_hbm.at[idx_v.at[pl.ds(off, rows)]],
            rows_v.at[slot].at[pl.ds(0, rows)],
            gsems[slot])
        ocopy = pltpu.async_copy(
            outputs_hbm.at[pl.ds(base + off, rows)],
            outs_v.at[slot].at[pl.ds(0, rows)],
            osems[slot])
        return gather, ocopy

    pending = {g: start(g) for g in range(NBUF)}

    accs = tuple(jnp.zeros((L,), jnp.float32) for _ in range(C_VECS))
    for g in range(len(CHUNKS)):
        slot = g % NBUF
        rows = CHUNKS[g][1]
        gather, ocopy = pending.pop(g)
        gather.wait()
        ocopy.wait()

        def body(r, accs, slot=slot):
            return tuple(
                accs[cc]
                + rows_v[slot, r, pl.ds(cc * L, L)]
                * outs_v[slot, r, pl.ds(cc * L, L)]
                for cc in range(C_VECS)
            )
        accs = lax.fori_loop(0, rows, body, accs)
        if g + NBUF < len(CHUNKS):
            pending[g + NBUF] = start(g + NBUF)

    total = accs[0]
    for cc in range(1, C_VECS):
        total = total + accs[cc]
    acc_v[...] = total
    pltpu.sync_copy(acc_v, out_hbm.at[wid])


TC_BLOCK = 8192
N_TC_BLOCKS = B // TC_BLOCK


def _tc_body(x_ref, out_ref):
    i = pl.program_id(0)
    x = x_ref[...]
    s = jnp.sum(jnp.maximum(x, 0.0) + jnp.log1p(jnp.exp(-jnp.abs(x))))

    @pl.when(i == 0)
    def _init():
        out_ref[0, 0] = 0.0

    out_ref[0, 0] += s


_tc_dense = pl.pallas_call(
    _tc_body,
    grid=(N_TC_BLOCKS,),
    in_specs=[pl.BlockSpec((TC_BLOCK, C), lambda i: (i, 0))],
    out_specs=pl.BlockSpec(memory_space=pltpu.SMEM),
    out_shape=jax.ShapeDtypeStruct((1, 1), jnp.float32),
)


def _tc_combine_body(dense_ref, part_ref, out_ref):
    out_ref[0, 0] = (dense_ref[0, 0] - jnp.sum(part_ref[...])) * (1.0 / (B * C))


_tc_combine = pl.pallas_call(
    _tc_combine_body,
    in_specs=[
        pl.BlockSpec(memory_space=pltpu.SMEM),
        pl.BlockSpec((NW, L), lambda: (0, 0)),
    ],
    out_specs=pl.BlockSpec(memory_space=pltpu.SMEM),
    out_shape=jax.ShapeDtypeStruct((1, 1), jnp.float32),
)


def kernel(outputs, index, confidence):
    partials = _sc_gather_dot(outputs, index, confidence)
    dense = _tc_dense(outputs)
    return _tc_combine(dense, partials)[0, 0]
